# baseline (speedup 1.0000x reference)
import functools

import jax
import jax.numpy as jnp
from jax import lax
from jax.experimental import pallas as pl
from jax.experimental.pallas import tpu as pltpu
from jax.experimental.pallas import tpu_sc as plsc

N = 10000
E = 320000
L = 640000
D = 128

NBINS = 80
CBITS = 12
C = 4096
NT = 32
NLANE = 16
RROUND = 2
CAP = 48
PT = L // NT
PR = PT // RROUND
BATCH = 2000
SUBSEG = NLANE * CAP
SEGW = NT * RROUND * SUBSEG
CNTW = NBINS * NLANE
PB = 768
GB = 128
NGB = PB // GB
ACC_ROWS = C + 256
TPC = C // NLANE


def _prelu(x, w):
    return jnp.where(x >= 0, x, w * x)


def _mesh():
    return plsc.VectorSubcoreMesh(core_axis_name="c", subcore_axis_name="s")


def _bin_body(lg0_hbm, lg1_hbm, seg0_hbm, segl_hbm, cnt_hbm,
              st0, st1, rb0, rbl, nxt, sem):
    cid = lax.axis_index("c")
    sid = lax.axis_index("s")
    wid = sid * 2 + cid
    lane = lax.iota(jnp.int32, 16)
    base_in = wid * PT
    for r in range(RROUND):
        def zloop(i, _):
            nxt[pl.ds(i * 16, 16)] = jnp.zeros((16,), jnp.int32)
            return 0
        lax.fori_loop(0, NBINS, zloop, 0)

        for b in range(PR // BATCH):
            off = base_in + r * PR + b * BATCH
            pltpu.sync_copy(lg0_hbm.at[pl.ds(off, BATCH)], st0)
            pltpu.sync_copy(lg1_hbm.at[pl.ds(off, BATCH)], st1)

            def vloop(j, _):
                v1 = st1[pl.ds(j * 16, 16)]
                v0 = st0[pl.ds(j * 16, 16)]
                bn = lax.shift_right_logical(v1, CBITS)
                loc = jnp.bitwise_and(v1, C - 1)
                key = bn * NLANE + lane
                cur = plsc.load_gather(nxt, [key])
                ok = cur < CAP
                pos = key * CAP + cur
                plsc.store_scatter(rb0, [pos], v0, mask=ok)
                plsc.store_scatter(rbl, [pos], loc, mask=ok)
                plsc.store_scatter(nxt, [key], jnp.minimum(cur + 1, CAP))
                return 0
            lax.fori_loop(0, BATCH // 16, vloop, 0)

        cps = []
        for bf in range(NBINS):
            cps.append(pltpu.make_async_copy(
                rb0.at[pl.ds(bf * SUBSEG, SUBSEG)],
                seg0_hbm.at[bf, pl.ds((wid * RROUND + r) * SUBSEG, SUBSEG)],
                sem))
            cps.append(pltpu.make_async_copy(
                rbl.at[pl.ds(bf * SUBSEG, SUBSEG)],
                segl_hbm.at[bf, pl.ds((wid * RROUND + r) * SUBSEG, SUBSEG)],
                sem))
        cps.append(pltpu.make_async_copy(
            nxt, cnt_hbm.at[wid, pl.ds(r * CNTW, CNTW)], sem))
        for cp in cps:
            cp.start()
        for cp in cps:
            cp.wait()


def _bin_lg(lg0, lg1):
    kern = pl.kernel(
        _bin_body,
        mesh=_mesh(),
        compiler_params=pltpu.CompilerParams(needs_layout_passes=False),
        out_type=(
            jax.ShapeDtypeStruct((NBINS, SEGW), jnp.int32),
            jax.ShapeDtypeStruct((NBINS, SEGW), jnp.int32),
            jax.ShapeDtypeStruct((NT, RROUND * CNTW), jnp.int32),
        ),
        scratch_types=[
            pltpu.VMEM((BATCH,), jnp.int32),
            pltpu.VMEM((BATCH,), jnp.int32),
            pltpu.VMEM((NBINS * SUBSEG,), jnp.int32),
            pltpu.VMEM((NBINS * SUBSEG,), jnp.int32),
            pltpu.VMEM((CNTW,), jnp.int32),
            pltpu.SemaphoreType.DMA,
        ],
    )
    return kern(lg0, lg1)


def _lg_iter_body(seg0_hbm, segl_hbm, cnt_hbm, prev_hbm, zeros_hbm,
                  agg_hbm, acc_sh, seg0buf, seglbuf, cntb,
                  cst0_1d, cstl_1d, cst0_0, cst0_1, cst0_2, cst0_3, cst0_4,
                  cst0_5, cstl_0, cstl_1, cstl_2, cstl_3, cstl_4, cstl_5,
                  gbufA, gbufB, zbuf, semz, semg, sems, semo):
    cst0 = [cst0_0, cst0_1, cst0_2, cst0_3, cst0_4, cst0_5]
    cstl = [cstl_0, cstl_1, cstl_2, cstl_3, cstl_4, cstl_5]
    gbufs = [gbufA, gbufB]
    cid = lax.axis_index("c")
    sid = lax.axis_index("s")
    lane = lax.iota(jnp.int32, 16)

    pltpu.sync_copy(zeros_hbm, zbuf)
    for w2 in range(2):
        for r in range(RROUND):
            pltpu.sync_copy(
                cnt_hbm.at[2 * sid + w2, pl.ds(r * CNTW, CNTW)],
                cntb.at[pl.ds((w2 * RROUND + r) * CNTW, CNTW)])

    def chunk_body(k, _):
        c = cid * (NBINS // 2) + k
        gbase = c * C
        rbase = gbase + sid * TPC

        @pl.when(jnp.logical_and(k > 0, rbase - C < E))
        def _():
            pltpu.make_async_copy(
                acc_sh.at[pl.ds(sid * TPC, TPC)],
                agg_hbm.at[pl.ds(rbase - C, TPC)], semo).wait()
        zcps = [pltpu.async_copy(
            zbuf, acc_sh.at[pl.ds(sid * TPC + z * 32, 32)], semz)
            for z in range(TPC // 32)]
        zcps.append(pltpu.async_copy(
            zbuf.at[pl.ds(0, 16)], acc_sh.at[pl.ds(C + sid * 16, 16)], semz))

        pltpu.sync_copy(
            seg0_hbm.at[c, pl.ds(2 * sid * RROUND * SUBSEG,
                                 2 * RROUND * SUBSEG)], seg0buf)
        pltpu.sync_copy(
            segl_hbm.at[c, pl.ds(2 * sid * RROUND * SUBSEG,
                                 2 * RROUND * SUBSEG)], seglbuf)

        dummy0 = jnp.bitwise_and(lane, 7)
        dummyl = C + sid * 16 + jnp.bitwise_and(lane, 7)

        def pf(i, _):
            cst0_1d[pl.ds(i * 16, 16)] = dummy0
            cstl_1d[pl.ds(i * 16, 16)] = dummyl
            return 0
        lax.fori_loop(0, (PB + 16) // 16, pf, 0)

        def seg_loop(s, off):
            w2 = lax.shift_right_logical(s, 5)
            r = jnp.bitwise_and(lax.shift_right_logical(s, 4), 1)
            sl = jnp.bitwise_and(s, 15)
            cidx = (w2 * RROUND + r) * CNTW + c * NLANE + sl
            nb = plsc.load_gather(cntb, [jnp.full((16,), cidx, jnp.int32)])
            for j in range(CAP // 16):
                m = lane < (nb - j * 16)
                v0 = seg0buf[pl.ds(s * CAP + j * 16, 16)]
                vl = seglbuf[pl.ds(s * CAP + j * 16, 16)]
                plsc.store_compressed(cst0_1d.at[pl.ds(off, 16)], v0, mask=m)
                plsc.store_compressed(cstl_1d.at[pl.ds(off, 16)], vl, mask=m)
                off = off + plsc.all_reduce_population_count(m)[0]
            return off
        lax.fori_loop(0, 64, seg_loop, jnp.int32(0))

        for b in range(NGB):
            def cpy(j, _):
                cst0[b][pl.ds(j * 16, 16)] = cst0_1d[pl.ds(b * GB + j * 16, 16)]
                cstl[b][pl.ds(j * 16, 16)] = cstl_1d[pl.ds(b * GB + j * 16, 16)]
                return 0
            lax.fori_loop(0, GB // 16, cpy, 0)

        for cp in zcps:
            cp.wait()
        plsc.subcore_barrier()

        gcps = [None] * NGB
        scps = [None] * NGB
        gcps[0] = pltpu.async_copy(prev_hbm.at[cst0[0]], gbufs[0], semg)
        gcps[1] = pltpu.async_copy(prev_hbm.at[cst0[1]], gbufs[1], semg)
        for b in range(NGB):
            gcps[b].wait()
            scps[b] = pltpu.async_copy(gbufs[b % 2], acc_sh.at[cstl[b]],
                                       sems, add=True)
            if b + 2 < NGB:
                scps[b].wait()
                gcps[b + 2] = pltpu.async_copy(prev_hbm.at[cst0[b + 2]],
                                               gbufs[b % 2], semg)
        for b in range(max(NGB - 2, 0), NGB):
            scps[b].wait()
        plsc.subcore_barrier()

        @pl.when(rbase < E)
        def _():
            pltpu.async_copy(acc_sh.at[pl.ds(sid * TPC, TPC)],
                             agg_hbm.at[pl.ds(rbase, TPC)], semo)
        return 0

    lax.fori_loop(0, NBINS // 2, chunk_body, 0)
    rlast = (cid * (NBINS // 2) + NBINS // 2 - 1) * C + sid * TPC

    @pl.when(rlast < E)
    def _():
        pltpu.make_async_copy(acc_sh.at[pl.ds(sid * TPC, TPC)],
                              agg_hbm.at[pl.ds(rlast, TPC)], semo).wait()


def _lg_iter(seg0, segl, cnt, prev, zeros64):
    kern = pl.kernel(
        _lg_iter_body,
        mesh=_mesh(),
        compiler_params=pltpu.CompilerParams(needs_layout_passes=False),
        out_type=jax.ShapeDtypeStruct((E, D), jnp.float32),
        scratch_types=[
            pltpu.VMEM_SHARED((ACC_ROWS, D), jnp.float32),
            pltpu.VMEM((2 * RROUND * SUBSEG,), jnp.int32),
            pltpu.VMEM((2 * RROUND * SUBSEG,), jnp.int32),
            pltpu.VMEM((2 * RROUND * CNTW,), jnp.int32),
            pltpu.VMEM((PB + 16,), jnp.int32),
            pltpu.VMEM((PB + 16,), jnp.int32),
        ] + [pltpu.VMEM((GB,), jnp.int32) for _ in range(12)] + [
            pltpu.VMEM((GB, D), jnp.float32),
            pltpu.VMEM((GB, D), jnp.float32),
            pltpu.VMEM((32, D), jnp.float32),
            pltpu.SemaphoreType.DMA,
            pltpu.SemaphoreType.DMA,
            pltpu.SemaphoreType.DMA,
            pltpu.SemaphoreType.DMA,
        ],
    )
    return kern(seg0, segl, cnt, prev, zeros64)


def _upd_body(ea_ref, agg_ref, ew_ref, out_ref):
    out_ref[...] = ea_ref[...] + agg_ref[...] * ew_ref[...]


def _edge_update(edge_attr, agg, ew2d):
    blk = 2000
    grid = (E // blk,)
    spec = pl.BlockSpec((blk, D), lambda i: (i, 0))
    wspec = pl.BlockSpec((blk, 1), lambda i: (i, 0))
    return pl.pallas_call(
        _upd_body,
        grid=grid,
        in_specs=[spec, spec, wspec],
        out_specs=spec,
        out_shape=jax.ShapeDtypeStruct((E, D), jnp.float32),
    )(edge_attr, agg, ew2d)


XGB = 80
XNB = (E // NT) // XGB


def _xn_body(dst_hbm, out_hbm, zeros_hbm, part_hbm, acc_sh,
             idxA, idxB, gbufA, gbufB, zbuf, semz, semg, sems):
    cid = lax.axis_index("c")
    sid = lax.axis_index("s")
    idxs = [idxA, idxB]
    gbufs = [gbufA, gbufB]
    base = (cid * NLANE + sid) * (E // NT)

    pltpu.sync_copy(zeros_hbm, zbuf)
    zcps = [pltpu.async_copy(
        zbuf, acc_sh.at[pl.ds(sid * 624 + z * 128, 128)], semz)
        for z in range(5)]
    for cp in zcps:
        cp.wait()
    plsc.subcore_barrier()

    def batch(b, _):
        bb = jnp.bitwise_and(b, 1)
        pltpu.sync_copy(dst_hbm.at[pl.ds(base + b * XGB, XGB)], idxs[0])
        pltpu.async_copy(out_hbm.at[pl.ds(base + b * XGB, XGB)],
                         gbufs[0], semg).wait()
        pltpu.sync_copy(gbufs[0], acc_sh.at[idxs[0]], add=True)
        return 0
    lax.fori_loop(0, XNB, batch, 0)
    plsc.subcore_barrier()
    pltpu.sync_copy(acc_sh.at[pl.ds(sid * 624, 640)],
                    part_hbm.at[cid, pl.ds(sid * 624, 640)])


def _xn_scatter(dst, out):
    zeros125 = jnp.zeros((128, D), jnp.float32)
    kern = pl.kernel(
        _xn_body,
        mesh=_mesh(),
        compiler_params=pltpu.CompilerParams(needs_layout_passes=False),
        out_type=jax.ShapeDtypeStruct((2, N, D), jnp.float32),
        scratch_types=[
            pltpu.VMEM_SHARED((N, D), jnp.float32),
            pltpu.VMEM((XGB,), jnp.int32),
            pltpu.VMEM((XGB,), jnp.int32),
            pltpu.VMEM((XGB, D), jnp.float32),
            pltpu.VMEM((XGB, D), jnp.float32),
            pltpu.VMEM((128, D), jnp.float32),
            pltpu.SemaphoreType.DMA,
            pltpu.SemaphoreType.DMA,
            pltpu.SemaphoreType.DMA,
        ],
    )
    return kern(dst, out, zeros125)


def _alpha_body(gid_ref, gjs_ref, xs_ref, ef_ref, dg_ref, bias_ref, p_ref,
                sw_ref, sb_ref, we_ref, be_ref, ea_ref, ew_ref):
    g = gid_ref[...] + gjs_ref[...] + bias_ref[...]
    t = _prelu(g, p_ref[0, 0]) @ sw_ref[...] + sb_ref[...]
    ef = ef_ref[...] @ we_ref[...] + be_ref[...]
    a = (t * ef).sum(-1, keepdims=True) / dg_ref[...]
    ew = jax.nn.sigmoid(a)
    ea_ref[...] = xs_ref[...] * ew
    ew_ref[...] = ew


def _edge_gate(gid, gjs, xs, efeat, degs, bias, sml_p, sml_W, sml_b,
               edge_emb_W, edge_emb_b):
    blk = 2000
    grid = (E // blk,)
    full = lambda *_: (0, 0)
    spec = pl.BlockSpec((blk, D), lambda i: (i, 0))
    espec = pl.BlockSpec((blk, 16), lambda i: (i, 0))
    sspec = pl.BlockSpec((blk, 1), lambda i: (i, 0))
    return pl.pallas_call(
        _alpha_body,
        grid=grid,
        in_specs=[spec, spec, spec, espec, sspec,
                  pl.BlockSpec((1, D), full), pl.BlockSpec((1, 1), full),
                  pl.BlockSpec((D, D), full), pl.BlockSpec((1, D), full),
                  pl.BlockSpec((16, D), full), pl.BlockSpec((1, D), full)],
        out_specs=[spec, sspec],
        out_shape=(jax.ShapeDtypeStruct((E, D), jnp.float32),
                   jax.ShapeDtypeStruct((E, 1), jnp.float32)),
    )(gid, gjs, xs, efeat, degs, bias.reshape(1, D), sml_p.reshape(1, 1),
      sml_W, sml_b.reshape(1, D), edge_emb_W, edge_emb_b.reshape(1, D))


def _mlp_body(x_ref, pa_ref, pb_ref, w1_ref, b1_ref, p2_ref, w2_ref, b2_ref,
              p3_ref, w3_ref, b3_ref, p4_ref, w4_ref, b4_ref, out_ref):
    xn = x_ref[...] + pa_ref[...] + pb_ref[...]
    h = xn @ w1_ref[...] + b1_ref[...]
    h2 = _prelu(h, p2_ref[0, 0]) @ w2_ref[...] + b2_ref[...]
    h3 = _prelu(h2, p3_ref[0, 0]) @ w3_ref[...] + b3_ref[...]
    h = (h3 + h) * 0.5
    h4 = _prelu(h, p4_ref[0, 0]) @ w4_ref[...] + b4_ref[...]
    out_ref[...] = (h4 + h) * 0.5


def _mlp(x, pa, pb, lin1_W, lin1_b, lin2_p, lin2_W, lin2_b, lin3_p, lin3_W,
         lin3_b, lin4_p, lin4_W, lin4_b):
    n, d = x.shape
    blk = 1000
    grid = (n // blk,)
    full = lambda *_: (0, 0)
    w_spec = pl.BlockSpec((d, d), full)
    b_spec = pl.BlockSpec((1, d), full)
    p_spec = pl.BlockSpec((1, 1), full)
    return pl.pallas_call(
        _mlp_body,
        grid=grid,
        in_specs=[
            pl.BlockSpec((blk, d), lambda i: (i, 0)),
            pl.BlockSpec((blk, d), lambda i: (i, 0)),
            pl.BlockSpec((blk, d), lambda i: (i, 0)),
            w_spec, b_spec, p_spec, w_spec, b_spec, p_spec, w_spec, b_spec,
            p_spec, w_spec, b_spec,
        ],
        out_specs=pl.BlockSpec((blk, d), lambda i: (i, 0)),
        out_shape=jax.ShapeDtypeStruct((n, d), jnp.float32),
    )(x, pa, pb, lin1_W, lin1_b.reshape(1, d), lin2_p.reshape(1, 1), lin2_W,
      lin2_b.reshape(1, d), lin3_p.reshape(1, 1), lin3_W, lin3_b.reshape(1, d),
      lin4_p.reshape(1, 1), lin4_W, lin4_b.reshape(1, d))


def kernel(x, xchemfea, edge_feats, edge_index, line_graph_edge_index, w_i,
           w_j, bias, edge_emb_W, edge_emb_b, sml_p, sml_W, sml_b, lin1_W,
           lin1_b, lin2_p, lin2_W, lin2_b, lin3_p, lin3_W, lin3_b, lin4_p,
           lin4_W, lin4_b):
    src = edge_index[0]
    dst = edge_index[1]
    deg = jnp.zeros((x.shape[0],), x.dtype).at[dst].add(1.0)
    alpha_i = x @ w_i
    alpha_j = x @ w_j
    degs = deg[src].reshape(E, 1)
    edge_attr, ew2d = _edge_gate(alpha_i[dst], alpha_j[src], x[src],
                                 edge_feats, degs, bias, sml_p, sml_W, sml_b,
                                 edge_emb_W, edge_emb_b)
    ew = ew2d.reshape(E)

    lg0 = line_graph_edge_index[0]
    lg1 = line_graph_edge_index[1]
    seg0, segl, cnt = _bin_lg(lg0, lg1)
    zeros64 = jnp.zeros((32, D), jnp.float32)
    out = edge_attr
    for _ in range(3):
        agg = _lg_iter(seg0, segl, cnt, out, zeros64)
        out = _edge_update(edge_attr, agg, ew2d)

    part = _xn_scatter(dst, out)
    return _mlp(x, part[0], part[1], lin1_W, lin1_b, lin2_p, lin2_W, lin2_b,
                lin3_p, lin3_W, lin3_b, lin4_p, lin4_W, lin4_b)


# speedup vs baseline: 1.0245x; 1.0245x over previous
import functools

import jax
import jax.numpy as jnp
from jax import lax
from jax.experimental import pallas as pl
from jax.experimental.pallas import tpu as pltpu
from jax.experimental.pallas import tpu_sc as plsc

N = 10000
E = 320000
L = 640000
D = 128

NBINS = 80
CBITS = 12
C = 4096
NT = 32
NLANE = 16
RROUND = 2
CAP = 48
PT = L // NT
PR = PT // RROUND
BATCH = 2000
SUBSEG = NLANE * CAP
SEGW = NT * RROUND * SUBSEG
CNTW = NBINS * NLANE
PB = 768
GB = 128
NGB = PB // GB
ACC_ROWS = C + 256
TPC = C // NLANE


def _prelu(x, w):
    return jnp.where(x >= 0, x, w * x)


def _mesh():
    return plsc.VectorSubcoreMesh(core_axis_name="c", subcore_axis_name="s")


def _bin_body(lg0_hbm, lg1_hbm, seg0_hbm, segl_hbm, cnt_hbm,
              st0, st1, rb0, rbl, nxt, sem):
    cid = lax.axis_index("c")
    sid = lax.axis_index("s")
    wid = sid * 2 + cid
    lane = lax.iota(jnp.int32, 16)
    base_in = wid * PT
    for r in range(RROUND):
        def zloop(i, _):
            nxt[pl.ds(i * 16, 16)] = jnp.zeros((16,), jnp.int32)
            return 0
        lax.fori_loop(0, NBINS, zloop, 0)

        for b in range(PR // BATCH):
            off = base_in + r * PR + b * BATCH
            pltpu.sync_copy(lg0_hbm.at[pl.ds(off, BATCH)], st0)
            pltpu.sync_copy(lg1_hbm.at[pl.ds(off, BATCH)], st1)

            def vloop(j, _):
                v1 = st1[pl.ds(j * 16, 16)]
                v0 = st0[pl.ds(j * 16, 16)]
                bn = lax.shift_right_logical(v1, CBITS)
                loc = jnp.bitwise_and(v1, C - 1)
                key = bn * NLANE + lane
                cur = plsc.load_gather(nxt, [key])
                ok = cur < CAP
                pos = key * CAP + cur
                plsc.store_scatter(rb0, [pos], v0, mask=ok)
                plsc.store_scatter(rbl, [pos], loc, mask=ok)
                plsc.store_scatter(nxt, [key], jnp.minimum(cur + 1, CAP))
                return 0
            lax.fori_loop(0, BATCH // 16, vloop, 0)

        cps = []
        for bf in range(NBINS):
            cps.append(pltpu.make_async_copy(
                rb0.at[pl.ds(bf * SUBSEG, SUBSEG)],
                seg0_hbm.at[bf, pl.ds((wid * RROUND + r) * SUBSEG, SUBSEG)],
                sem))
            cps.append(pltpu.make_async_copy(
                rbl.at[pl.ds(bf * SUBSEG, SUBSEG)],
                segl_hbm.at[bf, pl.ds((wid * RROUND + r) * SUBSEG, SUBSEG)],
                sem))
        cps.append(pltpu.make_async_copy(
            nxt, cnt_hbm.at[wid, pl.ds(r * CNTW, CNTW)], sem))
        for cp in cps:
            cp.start()
        for cp in cps:
            cp.wait()


def _bin_lg(lg0, lg1):
    kern = pl.kernel(
        _bin_body,
        mesh=_mesh(),
        compiler_params=pltpu.CompilerParams(needs_layout_passes=False),
        out_type=(
            jax.ShapeDtypeStruct((NBINS, SEGW), jnp.int32),
            jax.ShapeDtypeStruct((NBINS, SEGW), jnp.int32),
            jax.ShapeDtypeStruct((NT, RROUND * CNTW), jnp.int32),
        ),
        scratch_types=[
            pltpu.VMEM((BATCH,), jnp.int32),
            pltpu.VMEM((BATCH,), jnp.int32),
            pltpu.VMEM((NBINS * SUBSEG,), jnp.int32),
            pltpu.VMEM((NBINS * SUBSEG,), jnp.int32),
            pltpu.VMEM((CNTW,), jnp.int32),
            pltpu.SemaphoreType.DMA,
        ],
    )
    return kern(lg0, lg1)


def _lg_iter_body(seg0_hbm, segl_hbm, cnt_hbm, prev_hbm, zeros_hbm,
                  agg_hbm, acc_sh, seg0buf, seglbuf, cntb,
                  cst0_1d, cstl_1d, cst0_0, cst0_1, cst0_2, cst0_3, cst0_4,
                  cst0_5, cstl_0, cstl_1, cstl_2, cstl_3, cstl_4, cstl_5,
                  gbufA, gbufB, zbuf, semz, semg, sems, semo):
    cst0 = [cst0_0, cst0_1, cst0_2, cst0_3, cst0_4, cst0_5]
    cstl = [cstl_0, cstl_1, cstl_2, cstl_3, cstl_4, cstl_5]
    gbufs = [gbufA, gbufB]
    cid = lax.axis_index("c")
    sid = lax.axis_index("s")
    lane = lax.iota(jnp.int32, 16)

    pltpu.sync_copy(zeros_hbm, zbuf)
    for w2 in range(2):
        for r in range(RROUND):
            pltpu.sync_copy(
                cnt_hbm.at[2 * sid + w2, pl.ds(r * CNTW, CNTW)],
                cntb.at[pl.ds((w2 * RROUND + r) * CNTW, CNTW)])

    def chunk_body(k, _):
        c = cid * (NBINS // 2) + k
        gbase = c * C
        rbase = gbase + sid * TPC

        @pl.when(jnp.logical_and(k > 0, rbase - C < E))
        def _():
            pltpu.make_async_copy(
                acc_sh.at[pl.ds(sid * TPC, TPC)],
                agg_hbm.at[pl.ds(rbase - C, TPC)], semo).wait()
        zcps = [pltpu.async_copy(
            zbuf, acc_sh.at[pl.ds(sid * TPC + z * 32, 32)], semz)
            for z in range(TPC // 32)]
        zcps.append(pltpu.async_copy(
            zbuf.at[pl.ds(0, 16)], acc_sh.at[pl.ds(C + sid * 16, 16)], semz))

        pltpu.sync_copy(
            seg0_hbm.at[c, pl.ds(2 * sid * RROUND * SUBSEG,
                                 2 * RROUND * SUBSEG)], seg0buf)
        pltpu.sync_copy(
            segl_hbm.at[c, pl.ds(2 * sid * RROUND * SUBSEG,
                                 2 * RROUND * SUBSEG)], seglbuf)

        dummy0 = jnp.bitwise_and(lane, 7)
        dummyl = C + sid * 16 + jnp.bitwise_and(lane, 7)

        def pf(i, _):
            cst0_1d[pl.ds(i * 16, 16)] = dummy0
            cstl_1d[pl.ds(i * 16, 16)] = dummyl
            return 0
        lax.fori_loop(0, (PB + 16) // 16, pf, 0)

        def seg_loop(s, off):
            w2 = lax.shift_right_logical(s, 5)
            r = jnp.bitwise_and(lax.shift_right_logical(s, 4), 1)
            sl = jnp.bitwise_and(s, 15)
            cidx = (w2 * RROUND + r) * CNTW + c * NLANE + sl
            nb = plsc.load_gather(cntb, [jnp.full((16,), cidx, jnp.int32)])
            for j in range(CAP // 16):
                m = lane < (nb - j * 16)
                v0 = seg0buf[pl.ds(s * CAP + j * 16, 16)]
                vl = seglbuf[pl.ds(s * CAP + j * 16, 16)]
                plsc.store_compressed(cst0_1d.at[pl.ds(off, 16)], v0, mask=m)
                plsc.store_compressed(cstl_1d.at[pl.ds(off, 16)], vl, mask=m)
                off = off + plsc.all_reduce_population_count(m)[0]
            return off
        lax.fori_loop(0, 64, seg_loop, jnp.int32(0))

        for b in range(NGB):
            def cpy(j, _):
                cst0[b][pl.ds(j * 16, 16)] = cst0_1d[pl.ds(b * GB + j * 16, 16)]
                cstl[b][pl.ds(j * 16, 16)] = cstl_1d[pl.ds(b * GB + j * 16, 16)]
                return 0
            lax.fori_loop(0, GB // 16, cpy, 0)

        for cp in zcps:
            cp.wait()
        plsc.subcore_barrier()

        gcps = [None] * NGB
        scps = [None] * NGB
        gcps[0] = pltpu.async_copy(prev_hbm.at[cst0[0]], gbufs[0], semg)
        gcps[1] = pltpu.async_copy(prev_hbm.at[cst0[1]], gbufs[1], semg)
        for b in range(NGB):
            gcps[b].wait()
            scps[b] = pltpu.async_copy(gbufs[b % 2], acc_sh.at[cstl[b]],
                                       sems, add=True)
            if b + 2 < NGB:
                scps[b].wait()
                gcps[b + 2] = pltpu.async_copy(prev_hbm.at[cst0[b + 2]],
                                               gbufs[b % 2], semg)
        for b in range(max(NGB - 2, 0), NGB):
            scps[b].wait()
        plsc.subcore_barrier()

        @pl.when(rbase < E)
        def _():
            pltpu.async_copy(acc_sh.at[pl.ds(sid * TPC, TPC)],
                             agg_hbm.at[pl.ds(rbase, TPC)], semo)
        return 0

    lax.fori_loop(0, NBINS // 2, chunk_body, 0)
    rlast = (cid * (NBINS // 2) + NBINS // 2 - 1) * C + sid * TPC

    @pl.when(rlast < E)
    def _():
        pltpu.make_async_copy(acc_sh.at[pl.ds(sid * TPC, TPC)],
                              agg_hbm.at[pl.ds(rlast, TPC)], semo).wait()


def _lg_iter(seg0, segl, cnt, prev, zeros64):
    kern = pl.kernel(
        _lg_iter_body,
        mesh=_mesh(),
        compiler_params=pltpu.CompilerParams(needs_layout_passes=False),
        out_type=jax.ShapeDtypeStruct((E, D), jnp.float32),
        scratch_types=[
            pltpu.VMEM_SHARED((ACC_ROWS, D), jnp.float32),
            pltpu.VMEM((2 * RROUND * SUBSEG,), jnp.int32),
            pltpu.VMEM((2 * RROUND * SUBSEG,), jnp.int32),
            pltpu.VMEM((2 * RROUND * CNTW,), jnp.int32),
            pltpu.VMEM((PB + 16,), jnp.int32),
            pltpu.VMEM((PB + 16,), jnp.int32),
        ] + [pltpu.VMEM((GB,), jnp.int32) for _ in range(12)] + [
            pltpu.VMEM((GB, D), jnp.float32),
            pltpu.VMEM((GB, D), jnp.float32),
            pltpu.VMEM((32, D), jnp.float32),
            pltpu.SemaphoreType.DMA,
            pltpu.SemaphoreType.DMA,
            pltpu.SemaphoreType.DMA,
            pltpu.SemaphoreType.DMA,
        ],
    )
    return kern(seg0, segl, cnt, prev, zeros64)


def _upd_body(ea_ref, agg_ref, ew_ref, out_ref):
    out_ref[...] = ea_ref[...] + agg_ref[...] * ew_ref[...]


def _edge_update(edge_attr, agg, ew2d):
    blk = 2000
    grid = (E // blk,)
    spec = pl.BlockSpec((blk, D), lambda i: (i, 0))
    wspec = pl.BlockSpec((blk, 1), lambda i: (i, 0))
    return pl.pallas_call(
        _upd_body,
        grid=grid,
        in_specs=[spec, spec, wspec],
        out_specs=spec,
        out_shape=jax.ShapeDtypeStruct((E, D), jnp.float32),
    )(edge_attr, agg, ew2d)


XGB = 80
XNB = (E // NT) // XGB


def _xn_body(dst_hbm, out_hbm, zeros_hbm, part_hbm, acc_sh,
             idxA, idxB, gbufA, gbufB, zbuf, semz, semg, sems):
    cid = lax.axis_index("c")
    sid = lax.axis_index("s")
    idxs = [idxA, idxB]
    gbufs = [gbufA, gbufB]
    base = (cid * NLANE + sid) * (E // NT)

    pltpu.sync_copy(zeros_hbm, zbuf)
    zcps = [pltpu.async_copy(
        zbuf, acc_sh.at[pl.ds(sid * 624 + z * 128, 128)], semz)
        for z in range(5)]
    for cp in zcps:
        cp.wait()
    plsc.subcore_barrier()

    def batch(b, _):
        bb = jnp.bitwise_and(b, 1)
        pltpu.sync_copy(dst_hbm.at[pl.ds(base + b * XGB, XGB)], idxs[0])
        pltpu.async_copy(out_hbm.at[pl.ds(base + b * XGB, XGB)],
                         gbufs[0], semg).wait()
        pltpu.sync_copy(gbufs[0], acc_sh.at[idxs[0]], add=True)
        return 0
    lax.fori_loop(0, XNB, batch, 0)
    plsc.subcore_barrier()
    pltpu.sync_copy(acc_sh.at[pl.ds(sid * 624, 640)],
                    part_hbm.at[cid, pl.ds(sid * 624, 640)])


def _xn_scatter(dst, out):
    zeros125 = jnp.zeros((128, D), jnp.float32)
    kern = pl.kernel(
        _xn_body,
        mesh=_mesh(),
        compiler_params=pltpu.CompilerParams(needs_layout_passes=False),
        out_type=jax.ShapeDtypeStruct((2, N, D), jnp.float32),
        scratch_types=[
            pltpu.VMEM_SHARED((N, D), jnp.float32),
            pltpu.VMEM((XGB,), jnp.int32),
            pltpu.VMEM((XGB,), jnp.int32),
            pltpu.VMEM((XGB, D), jnp.float32),
            pltpu.VMEM((XGB, D), jnp.float32),
            pltpu.VMEM((128, D), jnp.float32),
            pltpu.SemaphoreType.DMA,
            pltpu.SemaphoreType.DMA,
            pltpu.SemaphoreType.DMA,
        ],
    )
    return kern(dst, out, zeros125)


def _alpha_body(gid_ref, gjs_ref, xs_ref, ef_ref, dg_ref, bias_ref, p_ref,
                sw_ref, sb_ref, we_ref, be_ref, ea_ref, ew_ref):
    g = gid_ref[...] + gjs_ref[...] + bias_ref[...]
    t = _prelu(g, p_ref[0, 0]) @ sw_ref[...] + sb_ref[...]
    ef = ef_ref[...] @ we_ref[...] + be_ref[...]
    a = (t * ef).sum(-1, keepdims=True) / dg_ref[...]
    ew = jax.nn.sigmoid(a)
    ea_ref[...] = xs_ref[...] * ew
    ew_ref[...] = ew


def _edge_gate(gid, gjs, xs, efeat, degs, bias, sml_p, sml_W, sml_b,
               edge_emb_W, edge_emb_b):
    blk = 2000
    grid = (E // blk,)
    full = lambda *_: (0, 0)
    spec = pl.BlockSpec((blk, D), lambda i: (i, 0))
    espec = pl.BlockSpec((blk, 16), lambda i: (i, 0))
    sspec = pl.BlockSpec((blk, 1), lambda i: (i, 0))
    return pl.pallas_call(
        _alpha_body,
        grid=grid,
        in_specs=[spec, spec, spec, espec, sspec,
                  pl.BlockSpec((1, D), full), pl.BlockSpec((1, 1), full),
                  pl.BlockSpec((D, D), full), pl.BlockSpec((1, D), full),
                  pl.BlockSpec((16, D), full), pl.BlockSpec((1, D), full)],
        out_specs=[spec, sspec],
        out_shape=(jax.ShapeDtypeStruct((E, D), jnp.float32),
                   jax.ShapeDtypeStruct((E, 1), jnp.float32)),
    )(gid, gjs, xs, efeat, degs, bias.reshape(1, D), sml_p.reshape(1, 1),
      sml_W, sml_b.reshape(1, D), edge_emb_W, edge_emb_b.reshape(1, D))


def _mlp_body(x_ref, pa_ref, pb_ref, w1_ref, b1_ref, p2_ref, w2_ref, b2_ref,
              p3_ref, w3_ref, b3_ref, p4_ref, w4_ref, b4_ref, out_ref):
    xn = x_ref[...] + pa_ref[...] + pb_ref[...]
    h = xn @ w1_ref[...] + b1_ref[...]
    h2 = _prelu(h, p2_ref[0, 0]) @ w2_ref[...] + b2_ref[...]
    h3 = _prelu(h2, p3_ref[0, 0]) @ w3_ref[...] + b3_ref[...]
    h = (h3 + h) * 0.5
    h4 = _prelu(h, p4_ref[0, 0]) @ w4_ref[...] + b4_ref[...]
    out_ref[...] = (h4 + h) * 0.5


def _mlp(x, pa, pb, lin1_W, lin1_b, lin2_p, lin2_W, lin2_b, lin3_p, lin3_W,
         lin3_b, lin4_p, lin4_W, lin4_b):
    n, d = x.shape
    blk = 1000
    grid = (n // blk,)
    full = lambda *_: (0, 0)
    w_spec = pl.BlockSpec((d, d), full)
    b_spec = pl.BlockSpec((1, d), full)
    p_spec = pl.BlockSpec((1, 1), full)
    return pl.pallas_call(
        _mlp_body,
        grid=grid,
        in_specs=[
            pl.BlockSpec((blk, d), lambda i: (i, 0)),
            pl.BlockSpec((blk, d), lambda i: (i, 0)),
            pl.BlockSpec((blk, d), lambda i: (i, 0)),
            w_spec, b_spec, p_spec, w_spec, b_spec, p_spec, w_spec, b_spec,
            p_spec, w_spec, b_spec,
        ],
        out_specs=pl.BlockSpec((blk, d), lambda i: (i, 0)),
        out_shape=jax.ShapeDtypeStruct((n, d), jnp.float32),
    )(x, pa, pb, lin1_W, lin1_b.reshape(1, d), lin2_p.reshape(1, 1), lin2_W,
      lin2_b.reshape(1, d), lin3_p.reshape(1, 1), lin3_W, lin3_b.reshape(1, d),
      lin4_p.reshape(1, 1), lin4_W, lin4_b.reshape(1, d))


def kernel(x, xchemfea, edge_feats, edge_index, line_graph_edge_index, w_i,
           w_j, bias, edge_emb_W, edge_emb_b, sml_p, sml_W, sml_b, lin1_W,
           lin1_b, lin2_p, lin2_W, lin2_b, lin3_p, lin3_W, lin3_b, lin4_p,
           lin4_W, lin4_b):
    src = edge_index[0]
    dst = edge_index[1]
    ef = edge_feats @ edge_emb_W + edge_emb_b
    deg = jnp.zeros((x.shape[0],), x.dtype).at[dst].add(1.0)
    alpha_i = x @ w_i
    alpha_j = x @ w_j
    alpha = alpha_i[dst] + alpha_j[src] + bias
    alpha = _prelu(alpha, sml_p) @ sml_W + sml_b
    alpha = (alpha * ef).sum(-1)
    alpha = alpha / deg[src]
    ew = jax.nn.sigmoid(alpha)
    edge_attr = x[src] * ew[:, None]

    lg0 = line_graph_edge_index[0]
    lg1 = line_graph_edge_index[1]
    seg0, segl, cnt = _bin_lg(lg0, lg1)
    zeros64 = jnp.zeros((32, D), jnp.float32)
    ew2d = ew.reshape(E, 1)
    out = edge_attr
    for _ in range(3):
        agg = _lg_iter(seg0, segl, cnt, out, zeros64)
        out = _edge_update(edge_attr, agg, ew2d)

    part = _xn_scatter(dst, out)
    return _mlp(x, part[0], part[1], lin1_W, lin1_b, lin2_p, lin2_W, lin2_b,
                lin3_p, lin3_W, lin3_b, lin4_p, lin4_W, lin4_b)
